# Initial kernel scaffold; baseline (speedup 1.0000x reference)
import jax
import jax.numpy as jnp
from jax.experimental import pallas as pl


def kernel(x, edge_index, batch, Wrel1, brel1, Wroot1, pw1, Wrel2, brel2, Wroot2, pw2, Wrel3, brel3, Wroot3, pw3, Wfc1, bfc1, Wfc2, bfc2, Wfc3, bfc3):
    raise NotImplementedError("write your pallas kernel here")


# speedup vs baseline: 12.0909x; 12.0909x over previous
import functools

import jax
import jax.numpy as jnp
from jax import lax
from jax.experimental import pallas as pl
from jax.experimental.pallas import tpu as pltpu
from jax.experimental.pallas import tpu_sc as plsc

N = 10000
NPAD = 10240
E = 320000
EPAD = 327680
NG = 16
NLANE = 128
NC = 2
NS = 16
NW = NC * NS
EPW = EPAD // NW
RPS = NPAD // NS
F32 = jnp.float32
I32 = jnp.int32


@functools.cache
def _make_spmm(D):
    mesh = plsc.VectorSubcoreMesh(core_axis_name="c", subcore_axis_name="s")

    @functools.partial(
        pl.kernel,
        mesh=mesh,
        compiler_params=pltpu.CompilerParams(use_tc_tiling_on_sc=False),
        out_type=jax.ShapeDtypeStruct((NC, NPAD, D), F32),
        scratch_types=[
            pltpu.VMEM((2, 8, 128), I32),
            pltpu.VMEM((128, D), F32),
            pltpu.SemaphoreType.DMA,
            pltpu.VMEM_SHARED((NPAD, D), F32),
        ],
    )
    def spmm(xin, srcr, dstr, zrows, out, ids, rows, sem, agg):
        c = lax.axis_index("c")
        s = lax.axis_index("s")
        w = c * NS + s
        pltpu.sync_copy(zrows, agg.at[pl.ds(s * RPS, RPS)])
        plsc.subcore_barrier()

        def chunk(i, carry):
            pltpu.sync_copy(srcr.at[w, i], ids.at[0])
            pltpu.sync_copy(dstr.at[w, i], ids.at[1])
            for j in range(8):
                pltpu.async_copy(xin.at[ids.at[0, j]], rows, sem).wait()
                pltpu.sync_copy(rows, agg.at[ids.at[1, j]], add=True)
            return carry

        lax.fori_loop(0, EPW // 1024, chunk, 0)
        plsc.subcore_barrier()
        pltpu.sync_copy(agg.at[pl.ds(s * RPS, RPS)],
                        out.at[c, pl.ds(s * RPS, RPS)])

    return spmm


def _dot(a, b):
    return jnp.dot(a, b, preferred_element_type=F32)


def _sortable(score):
    s = lax.bitcast_convert_type(score, I32)
    return jnp.where(s >= 0, s, s ^ I32(0x7FFFFFFF))


RC = (NPAD // NLANE, NLANE)


def _gcount(batch, maskb):
    laneiota = lax.broadcasted_iota(I32, (1, NLANE), 1)
    cnt = jnp.zeros((1, NLANE), F32)
    for g in range(NG):
        cg = jnp.sum(jnp.where((batch == I32(g)) & maskb, F32(1.0), F32(0.0)))
        cnt = jnp.where(laneiota == I32(g), cg, cnt)
    return cnt


def _gatherg(batch, vec, fill):
    out = jnp.full(RC, I32(fill))
    for g in range(NG):
        out = jnp.where(batch == I32(g), vec[0, g], out)
    return out


def _search(key, batch, need, lo0, hi0, iters):
    lo = jnp.full((1, NLANE), lo0, I32)
    hi = jnp.full((1, NLANE), hi0, I32)

    def body(i, c):
        lo, hi = c
        mid = (lo + hi) >> 1
        midv = _gatherg(batch, mid, 0x7FFFFFFF)
        cnt = _gcount(batch, key >= midv)
        ge = cnt >= need
        return (jnp.where(ge, mid, lo), jnp.where(ge, hi, mid))

    lo, hi = lax.fori_loop(0, iters, body, (lo, hi))
    return lo


def _topk(scores, batch, alive):
    aliveb = alive > F32(0.5)
    need = jnp.ceil(F32(0.8) * _gcount(batch, aliveb))

    keep = jnp.zeros(RC, jnp.bool_)
    cand = aliveb
    intmin = I32(-2**31)
    for s in scores:
        key = jnp.where(cand, _sortable(s), intmin)
        K = _search(key, batch, need, -0x40000000, 0x40000000, 31)
        Kv = _gatherg(batch, K, 0x7FFFFFFF)
        gt = key > Kv
        m = _gcount(batch, gt)
        keep = keep | gt
        cand = cand & (key == Kv)
        need = need - m
    negidx = -(lax.broadcasted_iota(I32, RC, 0) * I32(NLANE)
               + lax.broadcasted_iota(I32, RC, 1))
    key = jnp.where(cand, negidx, intmin)
    K = _search(key, batch, need, -0x40000000, 1, 31)
    keep = keep | (cand & (key >= _gatherg(batch, K, 0x7FFFFFFF)))
    return keep


_BROW = 1024


def _prep_body(aggp, xcur, wrelT, brel, wrootT, pw, pwn, h_out, score_out):
    agg = aggp[0] + aggp[1]
    h = jnp.maximum(_dot(agg, wrelT[...]) + brel[...] + _dot(xcur[...], wrootT[...]), F32(0.0))
    u = _dot(h, pw[...]) / pwn[...]
    h_out[...] = h
    score_out[...] = jnp.tanh(u)


def _make_prep(F):
    grid = (NPAD // _BROW,)
    return pl.pallas_call(
        _prep_body,
        grid=grid,
        in_specs=[
            pl.BlockSpec((2, _BROW, F), lambda i: (0, i, 0)),
            pl.BlockSpec((_BROW, F), lambda i: (i, 0)),
            pl.BlockSpec((F, 64), lambda i: (0, 0)),
            pl.BlockSpec((1, 64), lambda i: (0, 0)),
            pl.BlockSpec((F, 64), lambda i: (0, 0)),
            pl.BlockSpec((64, 1), lambda i: (0, 0)),
            pl.BlockSpec((1, 1), lambda i: (0, 0)),
        ],
        out_specs=[
            pl.BlockSpec((_BROW, 64), lambda i: (i, 0)),
            pl.BlockSpec((_BROW, 1), lambda i: (i, 0)),
        ],
        out_shape=[
            jax.ShapeDtypeStruct((NPAD, 64), F32),
            jax.ShapeDtypeStruct((NPAD, 1), F32),
        ],
    )


_prep128 = _make_prep(128)
_prep64 = _make_prep(64)


def _make_topk(nprev):
    def body(*refs):
        score, prev, (batchrc, alive, keep_out) = refs[0], refs[1:1 + nprev], refs[1 + nprev:]
        scores = [score[...]] + [p[...] for p in prev]
        keep = _topk(scores, batchrc[...], alive[...])
        keep_out[...] = jnp.where(keep, F32(1.0), F32(0.0))

    return pl.pallas_call(body, out_shape=jax.ShapeDtypeStruct(RC, F32))


_topk1 = _make_topk(0)
_topk2 = _make_topk(1)
_topk3 = _make_topk(2)


def _scale_body(h, score, keep, x_out):
    x_out[...] = h[...] * score[...] * keep[...]


_scale = pl.pallas_call(
    _scale_body,
    grid=(NPAD // _BROW,),
    in_specs=[
        pl.BlockSpec((_BROW, 64), lambda i: (i, 0)),
        pl.BlockSpec((_BROW, 1), lambda i: (i, 0)),
        pl.BlockSpec((_BROW, 1), lambda i: (i, 0)),
    ],
    out_specs=pl.BlockSpec((_BROW, 64), lambda i: (i, 0)),
    out_shape=jax.ShapeDtypeStruct((NPAD, 64), F32),
)


def _final_body(x4, alive, batch2, batchT, w1T, b1, w2T, b2, w3T, b3, out):
    giota_c = lax.broadcasted_iota(I32, (NLANE, 1), 0)
    onehotT = jnp.where(batchT[...] == giota_c, F32(1.0), F32(0.0))
    cnt = _dot(onehotT, alive[...])
    ssum = _dot(onehotT, x4[...])
    mean = (ssum / jnp.maximum(cnt, F32(1.0)))[:NG]
    aliveb = alive[...] > F32(0.5)
    xv = x4[...]
    b2d = batch2[...]
    ninf = F32(-jnp.inf)
    rows = []
    for g in range(NG):
        mask = (b2d == I32(g)) & aliveb
        rows.append(jnp.max(jnp.where(mask, xv, ninf), axis=0, keepdims=True))
    mx = jnp.concatenate(rows, axis=0)
    mx = jnp.where(cnt[:NG] > F32(0.0), mx, F32(0.0))
    g16 = jnp.concatenate([mean, mx], axis=1)
    g16 = jnp.maximum(_dot(g16, w1T[...]) + b1[...], F32(0.0))
    g16 = jnp.maximum(_dot(g16, w2T[...]) + b2[...], F32(0.0))
    logits = _dot(g16, w3T[...]) + b3[...]
    shifted = logits - jnp.max(logits, axis=1, keepdims=True)
    out[...] = shifted - jnp.log(jnp.sum(jnp.exp(shifted), axis=1, keepdims=True))


_final = pl.pallas_call(_final_body, out_shape=jax.ShapeDtypeStruct((NG, 10), F32))


def kernel(x, edge_index, batch, Wrel1, brel1, Wroot1, pw1, Wrel2, brel2,
           Wroot2, pw2, Wrel3, brel3, Wroot3, pw3, Wfc1, bfc1, Wfc2, bfc2,
           Wfc3, bfc3):
    xp = jnp.pad(x, ((0, NPAD - N), (0, 0)))
    batchp = jnp.concatenate([batch, jnp.full((NPAD - N,), NG, I32)])
    b2 = batchp[:, None]
    bT = batchp[None, :]
    epad = jnp.full((EPAD - E,), NPAD - 1, I32)
    srcr = jnp.concatenate([edge_index[0], epad]).reshape(NW, EPW // 1024, 8, 128)
    dstr = jnp.concatenate([edge_index[1], epad]).reshape(NW, EPW // 1024, 8, 128)
    z128 = jnp.zeros((RPS, 128), F32)
    z64 = jnp.zeros((RPS, 64), F32)
    alive0 = (jnp.arange(NPAD) < N).astype(F32).reshape(RC)

    def stage_args(Wrel, brel, Wroot, pw):
        return (Wrel.T, brel[None, :], Wroot.T, pw[:, None],
                jnp.linalg.norm(pw)[None, None])

    brc = batchp.reshape(RC)

    aggp = _make_spmm(128)(xp, srcr, dstr, z128)
    h1, s1 = _prep128(aggp, xp, *stage_args(Wrel1, brel1, Wroot1, pw1))
    a1 = _topk1(s1.reshape(RC), brc, alive0)
    x2 = _scale(h1, s1, a1.reshape(NPAD, 1))

    aggp = _make_spmm(64)(x2, srcr, dstr, z64)
    h2, s2 = _prep64(aggp, x2, *stage_args(Wrel2, brel2, Wroot2, pw2))
    a2 = _topk2(s2.reshape(RC), s1.reshape(RC), brc, a1)
    x3 = _scale(h2, s2, a2.reshape(NPAD, 1))

    aggp = _make_spmm(64)(x3, srcr, dstr, z64)
    h3, s3 = _prep64(aggp, x3, *stage_args(Wrel3, brel3, Wroot3, pw3))
    a3 = _topk3(s3.reshape(RC), s2.reshape(RC), s1.reshape(RC), brc, a2)
    x4 = _scale(h3, s3, a3.reshape(NPAD, 1))

    return _final(x4, a3.reshape(NPAD, 1), b2, bT, Wfc1.T, bfc1[None, :],
                  Wfc2.T, bfc2[None, :], Wfc3.T, bfc3[None, :])


# speedup vs baseline: 12.7869x; 1.0576x over previous
import functools

import jax
import jax.numpy as jnp
from jax import lax
from jax.experimental import pallas as pl
from jax.experimental.pallas import tpu as pltpu
from jax.experimental.pallas import tpu_sc as plsc

N = 10000
NPAD = 10240
E = 320000
EPAD = 327680
NG = 16
NLANE = 128
NC = 2
NS = 16
NW = NC * NS
EPW = EPAD // NW
RPS = NPAD // NS
F32 = jnp.float32
I32 = jnp.int32


@functools.cache
def _make_spmm(D):
    mesh = plsc.VectorSubcoreMesh(core_axis_name="c", subcore_axis_name="s")

    JR = 1 if D == 128 else 4
    NCH = EPW // (JR * 128)

    @functools.partial(
        pl.kernel,
        mesh=mesh,
        compiler_params=pltpu.CompilerParams(use_tc_tiling_on_sc=False),
        out_type=jax.ShapeDtypeStruct((NC, NPAD, D), F32),
        scratch_types=[
            pltpu.VMEM((2, 2, JR, 128), I32),
            pltpu.VMEM((2, JR, 128, D), F32),
            pltpu.SemaphoreType.DMA,
            pltpu.SemaphoreType.DMA,
            pltpu.SemaphoreType.DMA,
            pltpu.VMEM_SHARED((NPAD, D), F32),
        ],
    )
    def spmm(xin, srcr, dstr, zrows, out, ids, rows, gsem, ssem0, ssem1, agg):
        c = lax.axis_index("c")
        s = lax.axis_index("s")
        w = c * NS + s
        pltpu.sync_copy(zrows, agg.at[pl.ds(s * RPS, RPS)])
        plsc.subcore_barrier()

        def half(i, g, ssem, first):
            if not first:
                for j in range(JR):
                    pltpu.make_async_copy(rows.at[g, j],
                                          agg.at[ids.at[g, 1, j]], ssem).wait()
            pltpu.sync_copy(srcr.at[w, i], ids.at[g, 0])
            pltpu.sync_copy(dstr.at[w, i], ids.at[g, 1])
            copies = [pltpu.async_copy(xin.at[ids.at[g, 0, j]],
                                       rows.at[g, j], gsem)
                      for j in range(JR)]
            for cp in copies:
                cp.wait()
            for j in range(JR):
                pltpu.async_copy(rows.at[g, j], agg.at[ids.at[g, 1, j]],
                                 ssem, add=True)

        half(0, 0, ssem0, True)
        half(1, 1, ssem1, True)

        def chunk(i, carry):
            half(2 * i + 2, 0, ssem0, False)
            half(2 * i + 3, 1, ssem1, False)
            return carry

        lax.fori_loop(0, NCH // 2 - 1, chunk, 0)
        for j in range(JR):
            pltpu.make_async_copy(rows.at[0, j], agg.at[ids.at[0, 1, j]],
                                  ssem0).wait()
            pltpu.make_async_copy(rows.at[1, j], agg.at[ids.at[1, 1, j]],
                                  ssem1).wait()
        plsc.subcore_barrier()
        pltpu.sync_copy(agg.at[pl.ds(s * RPS, RPS)],
                        out.at[c, pl.ds(s * RPS, RPS)])

    return spmm


def _dot(a, b):
    return jnp.dot(a, b, preferred_element_type=F32)


def _sortable(score):
    s = lax.bitcast_convert_type(score, I32)
    return jnp.where(s >= 0, s, s ^ I32(0x7FFFFFFF))


RC = (NPAD // NLANE, NLANE)


def _gcount(batch, maskb):
    laneiota = lax.broadcasted_iota(I32, (1, NLANE), 1)
    cnt = jnp.zeros((1, NLANE), F32)
    for g in range(NG):
        cg = jnp.sum(jnp.where((batch == I32(g)) & maskb, F32(1.0), F32(0.0)))
        cnt = jnp.where(laneiota == I32(g), cg, cnt)
    return cnt


def _gatherg(batch, vec, fill):
    out = jnp.full(RC, I32(fill))
    for g in range(NG):
        out = jnp.where(batch == I32(g), vec[0, g], out)
    return out


def _search(key, batch, need, lo0, hi0, iters):
    lo = jnp.full((1, NLANE), lo0, I32)
    hi = jnp.full((1, NLANE), hi0, I32)

    def body(i, c):
        lo, hi = c
        mid = (lo + hi) >> 1
        midv = _gatherg(batch, mid, 0x7FFFFFFF)
        cnt = _gcount(batch, key >= midv)
        ge = cnt >= need
        return (jnp.where(ge, mid, lo), jnp.where(ge, hi, mid))

    lo, hi = lax.fori_loop(0, iters, body, (lo, hi))
    return lo


def _topk(scores, batch, alive):
    aliveb = alive > F32(0.5)
    need = jnp.ceil(F32(0.8) * _gcount(batch, aliveb))

    keep = jnp.zeros(RC, jnp.bool_)
    cand = aliveb
    intmin = I32(-2**31)
    for s in scores:
        key = jnp.where(cand, _sortable(s), intmin)
        K = _search(key, batch, need, -0x40000000, 0x40000000, 31)
        Kv = _gatherg(batch, K, 0x7FFFFFFF)
        gt = key > Kv
        m = _gcount(batch, gt)
        keep = keep | gt
        cand = cand & (key == Kv)
        need = need - m
    negidx = -(lax.broadcasted_iota(I32, RC, 0) * I32(NLANE)
               + lax.broadcasted_iota(I32, RC, 1))
    key = jnp.where(cand, negidx, intmin)
    K = _search(key, batch, need, -0x40000000, 1, 31)
    keep = keep | (cand & (key >= _gatherg(batch, K, 0x7FFFFFFF)))
    return keep


_BROW = 1024


def _prep_body(aggp, xcur, wrelT, brel, wrootT, pw, pwn, h_out, score_out):
    agg = aggp[0] + aggp[1]
    h = jnp.maximum(_dot(agg, wrelT[...]) + brel[...] + _dot(xcur[...], wrootT[...]), F32(0.0))
    u = _dot(h, pw[...]) / pwn[...]
    h_out[...] = h
    score_out[...] = jnp.tanh(u)


def _make_prep(F):
    grid = (NPAD // _BROW,)
    return pl.pallas_call(
        _prep_body,
        grid=grid,
        in_specs=[
            pl.BlockSpec((2, _BROW, F), lambda i: (0, i, 0)),
            pl.BlockSpec((_BROW, F), lambda i: (i, 0)),
            pl.BlockSpec((F, 64), lambda i: (0, 0)),
            pl.BlockSpec((1, 64), lambda i: (0, 0)),
            pl.BlockSpec((F, 64), lambda i: (0, 0)),
            pl.BlockSpec((64, 1), lambda i: (0, 0)),
            pl.BlockSpec((1, 1), lambda i: (0, 0)),
        ],
        out_specs=[
            pl.BlockSpec((_BROW, 64), lambda i: (i, 0)),
            pl.BlockSpec((_BROW, 1), lambda i: (i, 0)),
        ],
        out_shape=[
            jax.ShapeDtypeStruct((NPAD, 64), F32),
            jax.ShapeDtypeStruct((NPAD, 1), F32),
        ],
    )


_prep128 = _make_prep(128)
_prep64 = _make_prep(64)


def _make_topk(nprev):
    def body(*refs):
        score, prev, (batchrc, alive, keep_out) = refs[0], refs[1:1 + nprev], refs[1 + nprev:]
        scores = [score[...]] + [p[...] for p in prev]
        keep = _topk(scores, batchrc[...], alive[...])
        keep_out[...] = jnp.where(keep, F32(1.0), F32(0.0))

    return pl.pallas_call(body, out_shape=jax.ShapeDtypeStruct(RC, F32))


_topk1 = _make_topk(0)
_topk2 = _make_topk(1)
_topk3 = _make_topk(2)


def _scale_body(h, score, keep, x_out):
    x_out[...] = h[...] * score[...] * keep[...]


_scale = pl.pallas_call(
    _scale_body,
    grid=(NPAD // _BROW,),
    in_specs=[
        pl.BlockSpec((_BROW, 64), lambda i: (i, 0)),
        pl.BlockSpec((_BROW, 1), lambda i: (i, 0)),
        pl.BlockSpec((_BROW, 1), lambda i: (i, 0)),
    ],
    out_specs=pl.BlockSpec((_BROW, 64), lambda i: (i, 0)),
    out_shape=jax.ShapeDtypeStruct((NPAD, 64), F32),
)


def _final_body(x4, alive, batch2, batchT, w1T, b1, w2T, b2, w3T, b3, out):
    giota_c = lax.broadcasted_iota(I32, (NLANE, 1), 0)
    onehotT = jnp.where(batchT[...] == giota_c, F32(1.0), F32(0.0))
    cnt = _dot(onehotT, alive[...])
    ssum = _dot(onehotT, x4[...])
    mean = (ssum / jnp.maximum(cnt, F32(1.0)))[:NG]
    aliveb = alive[...] > F32(0.5)
    xv = x4[...]
    b2d = batch2[...]
    ninf = F32(-jnp.inf)
    rows = []
    for g in range(NG):
        mask = (b2d == I32(g)) & aliveb
        rows.append(jnp.max(jnp.where(mask, xv, ninf), axis=0, keepdims=True))
    mx = jnp.concatenate(rows, axis=0)
    mx = jnp.where(cnt[:NG] > F32(0.0), mx, F32(0.0))
    g16 = jnp.concatenate([mean, mx], axis=1)
    g16 = jnp.maximum(_dot(g16, w1T[...]) + b1[...], F32(0.0))
    g16 = jnp.maximum(_dot(g16, w2T[...]) + b2[...], F32(0.0))
    logits = _dot(g16, w3T[...]) + b3[...]
    shifted = logits - jnp.max(logits, axis=1, keepdims=True)
    out[...] = shifted - jnp.log(jnp.sum(jnp.exp(shifted), axis=1, keepdims=True))


_final = pl.pallas_call(_final_body, out_shape=jax.ShapeDtypeStruct((NG, 10), F32))


def kernel(x, edge_index, batch, Wrel1, brel1, Wroot1, pw1, Wrel2, brel2,
           Wroot2, pw2, Wrel3, brel3, Wroot3, pw3, Wfc1, bfc1, Wfc2, bfc2,
           Wfc3, bfc3):
    xp = jnp.pad(x, ((0, NPAD - N), (0, 0)))
    batchp = jnp.concatenate([batch, jnp.full((NPAD - N,), NG, I32)])
    b2 = batchp[:, None]
    bT = batchp[None, :]
    epad = jnp.full((EPAD - E,), NPAD - 1, I32)
    srcp = jnp.concatenate([edge_index[0], epad])
    dstp = jnp.concatenate([edge_index[1], epad])
    srcr128 = srcp.reshape(NW, EPW // 128, 1, 128)
    dstr128 = dstp.reshape(NW, EPW // 128, 1, 128)
    srcr64 = srcp.reshape(NW, EPW // 512, 4, 128)
    dstr64 = dstp.reshape(NW, EPW // 512, 4, 128)
    z128 = jnp.zeros((RPS, 128), F32)
    z64 = jnp.zeros((RPS, 64), F32)
    alive0 = (jnp.arange(NPAD) < N).astype(F32).reshape(RC)

    def stage_args(Wrel, brel, Wroot, pw):
        return (Wrel.T, brel[None, :], Wroot.T, pw[:, None],
                jnp.linalg.norm(pw)[None, None])

    brc = batchp.reshape(RC)

    aggp = _make_spmm(128)(xp, srcr128, dstr128, z128)
    h1, s1 = _prep128(aggp, xp, *stage_args(Wrel1, brel1, Wroot1, pw1))
    a1 = _topk1(s1.reshape(RC), brc, alive0)
    x2 = _scale(h1, s1, a1.reshape(NPAD, 1))

    aggp = _make_spmm(64)(x2, srcr64, dstr64, z64)
    h2, s2 = _prep64(aggp, x2, *stage_args(Wrel2, brel2, Wroot2, pw2))
    a2 = _topk2(s2.reshape(RC), s1.reshape(RC), brc, a1)
    x3 = _scale(h2, s2, a2.reshape(NPAD, 1))

    aggp = _make_spmm(64)(x3, srcr64, dstr64, z64)
    h3, s3 = _prep64(aggp, x3, *stage_args(Wrel3, brel3, Wroot3, pw3))
    a3 = _topk3(s3.reshape(RC), s2.reshape(RC), s1.reshape(RC), brc, a2)
    x4 = _scale(h3, s3, a3.reshape(NPAD, 1))

    return _final(x4, a3.reshape(NPAD, 1), b2, bT, Wfc1.T, bfc1[None, :],
                  Wfc2.T, bfc2[None, :], Wfc3.T, bfc3[None, :])
